# Initial kernel scaffold; baseline (speedup 1.0000x reference)
import jax
import jax.numpy as jnp
from jax.experimental import pallas as pl


def kernel(x, edge_index, edge_attr, batch, params):
    raise NotImplementedError("write your pallas kernel here")


# speedup vs baseline: 4.1602x; 4.1602x over previous
import functools

import jax
import jax.numpy as jnp
from jax import lax
from jax.experimental import pallas as pl
from jax.experimental.pallas import tpu as pltpu
from jax.experimental.pallas import tpu_sc as plsc

N = 50000
E = 800000
D = 100
DP = 128
G = 512
NL = 5
AV = 119
BV = 5

BN = 512
NB = 98
N_PAD = NB * BN
NCHUNK = 4
CHUNK = N_PAD // NCHUNK
AGG_ROWS = 12800
TRASH0 = CHUNK
NW = 32
SHARE = 25088
E_PAD = NW * SHARE
CAP = SHARE
TROWS = BV * N_PAD
KB = 128
EB = 4096
STG = 4352

_HI = lax.Precision.HIGHEST


def _bucketize_body(src_ref, dst_ref, attr_ref, gidx_ref, dl_ref, cnt_ref,
                    sbuf, dbuf, abuf,
                    stg0, std0, stg1, std1, stg2, std2, stg3, std3, cbuf):
    cidx = lax.axis_index("c")
    sidx = lax.axis_index("s")
    wid = sidx * 2 + cidx
    base = wid * SHARE
    lanes = lax.broadcasted_iota(jnp.int32, (16,), 0)
    stgs = ((stg0, std0), (stg1, std1), (stg2, std2), (stg3, std3))
    offs = [jnp.int32(0)] * NCHUNK
    fills = [jnp.int32(0)] * NCHUNK

    block_sizes = [EB] * (SHARE // EB)
    if SHARE % EB:
        block_sizes.append(SHARE % EB)
    boff = 0
    for bs in block_sizes:
        pltpu.sync_copy(src_ref.at[pl.ds(pl.multiple_of(base + boff, 8), bs)], sbuf.at[pl.ds(0, bs)])
        pltpu.sync_copy(dst_ref.at[pl.ds(pl.multiple_of(base + boff, 8), bs)], dbuf.at[pl.ds(0, bs)])
        pltpu.sync_copy(attr_ref.at[pl.ds(pl.multiple_of(base + boff, 8), bs)], abuf.at[pl.ds(0, bs)])

        def vbody(v, carry, boff=boff):
            sl = pl.ds(v * 16, 16)
            s = sbuf[sl]
            d = dbuf[sl]
            a = abuf[sl]
            pos = base + boff + v * 16 + lanes
            valid = pos < E
            c = ((d >= CHUNK).astype(jnp.int32)
                 + (d >= 2 * CHUNK).astype(jnp.int32)
                 + (d >= 3 * CHUNK).astype(jnp.int32))
            dl = d - c * CHUNK
            gx = a * N_PAD + s
            new = []
            for b in range(NCHUNK):
                ob = carry[b]
                m = jnp.logical_and(valid, c == b)
                pos = ob + plsc.cumsum(m.astype(jnp.int32)) - 1
                plsc.store_scatter(stgs[b][0], [pos], gx, mask=m)
                plsc.store_scatter(stgs[b][1], [pos], dl, mask=m)
                cnt = jnp.max(plsc.all_reduce_population_count(m))
                new.append(ob + cnt)
            return tuple(new)

        offs = list(lax.fori_loop(0, bs // 16, vbody, tuple(offs)))
        boff += bs

        for b in range(NCHUNK):
            sg, sd = stgs[b]
            rbase = (b * NW + wid) * CAP
            fill0 = fills[b]
            nfl = offs[b] >> 10

            def fl(j, _, sg=sg, sd=sd, rbase=rbase, fill0=fill0):
                pltpu.sync_copy(sg.at[pl.ds(j * 1024, 1024)],
                                gidx_ref.at[pl.ds(pl.multiple_of(rbase + fill0 + j * 1024, 8), 1024)])
                pltpu.sync_copy(sd.at[pl.ds(j * 1024, 1024)],
                                dl_ref.at[pl.ds(pl.multiple_of(rbase + fill0 + j * 1024, 8), 1024)])
                return 0

            lax.fori_loop(0, nfl, fl, 0)
            flushed = nfl << 10
            newoff = offs[b] - flushed
            nsh = (newoff + 15) >> 4

            def sh(j, _, sg=sg, sd=sd, flushed=flushed):
                sg[pl.ds(pl.multiple_of(j * 16, 8), 16)] = sg[pl.ds(pl.multiple_of(flushed + j * 16, 8), 16)]
                sd[pl.ds(pl.multiple_of(j * 16, 8), 16)] = sd[pl.ds(pl.multiple_of(flushed + j * 16, 8), 16)]
                return 0

            lax.fori_loop(0, nsh, sh, 0)
            fills[b] = fill0 + flushed
            offs[b] = newoff

    counts_vec = jnp.zeros((16,), jnp.int32)
    for b in range(NCHUNK):
        sg, sd = stgs[b]
        rbase = (b * NW + wid) * CAP
        fill0 = fills[b]
        total = fill0 + offs[b]
        nfl = (offs[b] + 15) >> 4

        def tf(j, _, sg=sg, sd=sd, rbase=rbase, fill0=fill0):
            pltpu.sync_copy(sg.at[pl.ds(pl.multiple_of(j * 16, 8), 16)],
                            gidx_ref.at[pl.ds(pl.multiple_of(rbase + fill0 + j * 16, 8), 16)])
            pltpu.sync_copy(sd.at[pl.ds(pl.multiple_of(j * 16, 8), 16)],
                            dl_ref.at[pl.ds(pl.multiple_of(rbase + fill0 + j * 16, 8), 16)])
            return 0

        lax.fori_loop(0, nfl, tf, 0)
        counts_vec = counts_vec + jnp.where(lanes == b, total, 0)
    cbuf[...] = counts_vec
    pltpu.sync_copy(cbuf, cnt_ref.at[pl.ds(pl.multiple_of(wid * 16, 8), 16)])


def _edge_body(t_ref, gidx_ref, dl_ref, cnt_ref, zeros_ref, agg_ref,
               gix, dli, rows, cnts, aggS, sem):
    cidx = lax.axis_index("c")
    sidx = lax.axis_index("s")
    lanes = lax.broadcasted_iota(jnp.int32, (16,), 0)
    pltpu.sync_copy(cnt_ref, cnts)
    for p in range(2):
        c = cidx * 2 + p
        pltpu.sync_copy(zeros_ref.at[pl.ds(pl.multiple_of(sidx * (AGG_ROWS // 16), 8), AGG_ROWS // 16)],
                        aggS.at[pl.ds(pl.multiple_of(sidx * (AGG_ROWS // 16), 8), AGG_ROWS // 16)])
        plsc.subcore_barrier()
        for wsub in range(2):
            w = sidx * 2 + wsub
            cvec = cnts[pl.ds(pl.multiple_of(w * 16, 8), 16)]
            n = jnp.max(jnp.where(lanes == c, cvec, 0))
            rbase = (c * NW + w) * CAP
            nblk = (n + KB - 1) // KB

            def blkbody(j, _, rbase=rbase, n=n):
                boff = j * KB
                pltpu.sync_copy(gidx_ref.at[pl.ds(pl.multiple_of(rbase + boff, 8), KB)], gix)
                pltpu.sync_copy(dl_ref.at[pl.ds(pl.multiple_of(rbase + boff, 8), KB)], dli)
                for v in range(KB // 16):
                    sl = pl.ds(v * 16, 16)
                    posv = boff + v * 16 + lanes
                    mv = posv < n
                    gix[sl] = jnp.where(mv, gix[sl], posv & 2047)
                    dli[sl] = jnp.where(mv, dli[sl], TRASH0 + (posv & 255))
                pltpu.async_copy(t_ref.at[gix], rows, sem).wait()
                pltpu.sync_copy(rows, aggS.at[dli], add=True)
                return 0

            lax.fori_loop(0, nblk, blkbody, 0)
        plsc.subcore_barrier()
        pltpu.sync_copy(aggS.at[pl.ds(pl.multiple_of(sidx * (CHUNK // 16), 8), CHUNK // 16)],
                        agg_ref.at[pl.ds(pl.multiple_of(c * CHUNK + sidx * (CHUNK // 16), 8), CHUNK // 16)])
        plsc.subcore_barrier()


def _embed_body(xcol_ref, bcol_ref, emb_ref, h_ref, ss_ref, cnt_ref):
    ohx = (xcol_ref[...] == lax.broadcasted_iota(jnp.int32, (BN, DP), 1).astype(jnp.float32))
    ohx = ohx.astype(jnp.float32)
    h = jnp.dot(ohx, emb_ref[...], preferred_element_type=jnp.float32,
                precision=_HI)
    h_ref[...] = h
    ohb = (bcol_ref[...] == lax.broadcasted_iota(jnp.int32, (BN, G), 1).astype(jnp.float32))
    ohb = ohb.astype(jnp.float32)
    ss = lax.dot_general(ohb, h, (((0,), (0,)), ((), ())),
                         preferred_element_type=jnp.float32, precision=_HI)
    cnt = lax.dot_general(ohb, jnp.ones((BN, 1), jnp.float32),
                          (((0,), (0,)), ((), ())),
                          preferred_element_type=jnp.float32, precision=_HI)

    @pl.when(pl.program_id(0) == 0)
    def _():
        ss_ref[...] = ss
        cnt_ref[...] = cnt

    @pl.when(pl.program_id(0) != 0)
    def _():
        ss_ref[...] += ss
        cnt_ref[...] += cnt


def _hl_body(h_ref, bcol_ref, vn_ref, hl_ref):
    ohb = (bcol_ref[...] == lax.broadcasted_iota(jnp.int32, (BN, G), 1).astype(jnp.float32))
    ohb = ohb.astype(jnp.float32)
    hl_ref[...] = h_ref[...] + jnp.dot(ohb, vn_ref[...],
                                       preferred_element_type=jnp.float32,
                                       precision=_HI)


def _tt_body(hl_ref, bond_ref, t_ref):
    t_ref[0] = jnp.maximum(hl_ref[...] + bond_ref[0], 0.0)


def _post_body(hl_ref, agg_ref, bcol_ref, w1_ref, c1_ref, w2_ref, c2_ref,
               h_ref, ss_ref, *, last):
    z = hl_ref[...] + agg_ref[...]
    z1 = jnp.maximum(jnp.dot(z, w1_ref[...],
                             preferred_element_type=jnp.float32,
                             precision=_HI) + c1_ref[...], 0.0)
    h2 = jnp.dot(z1, w2_ref[...], preferred_element_type=jnp.float32,
                 precision=_HI) + c2_ref[...]
    if not last:
        h2 = jnp.maximum(h2, 0.0)
    h_ref[...] = h2
    ohb = (bcol_ref[...] == lax.broadcasted_iota(jnp.int32, (BN, G), 1).astype(jnp.float32))
    ohb = ohb.astype(jnp.float32)
    ss = lax.dot_general(ohb, h2, (((0,), (0,)), ((), ())),
                         preferred_element_type=jnp.float32, precision=_HI)

    @pl.when(pl.program_id(0) == 0)
    def _():
        ss_ref[...] = ss

    @pl.when(pl.program_id(0) != 0)
    def _():
        ss_ref[...] += ss


def _vn_body(ss_ref, cnt_ref, vn_ref, w1_ref, c1_ref, w2_ref, c2_ref, out_ref):
    vt = ss_ref[...] + (cnt_ref[...] + 1.0) * vn_ref[...]
    t = jnp.maximum(jnp.dot(vt, w1_ref[...],
                            preferred_element_type=jnp.float32,
                            precision=_HI) + c1_ref[...], 0.0)
    out_ref[...] = jnp.maximum(jnp.dot(t, w2_ref[...],
                                       preferred_element_type=jnp.float32,
                                       precision=_HI) + c2_ref[...], 0.0)


def _head_body(ss_ref, cnt_ref, wp_ref, bp_ref, out_ref):
    hg = ss_ref[...] / jnp.maximum(cnt_ref[...], 1.0)
    out_ref[...] = jnp.dot(hg, wp_ref[...], preferred_element_type=jnp.float32,
                           precision=_HI) + bp_ref[...]


def _pad2(w, rows, cols):
    out = jnp.zeros((rows, cols), jnp.float32)
    return out.at[:w.shape[0], :w.shape[1]].set(w)


def _pad1(v, n):
    out = jnp.zeros((n,), jnp.float32)
    return out.at[:v.shape[0]].set(v)


def kernel(x, edge_index, edge_attr, batch, params):
    f32 = jnp.float32
    i32 = jnp.int32

    src = edge_index[0].astype(i32)
    dst = edge_index[1].astype(i32)
    attr = edge_attr.astype(i32)
    pad_e = E_PAD - E
    src_p = jnp.pad(src, (0, pad_e))
    dst_p = jnp.pad(dst, (0, pad_e))
    attr_p = jnp.pad(attr, (0, pad_e))

    pad_n = N_PAD - N
    xcol = jnp.pad(x.astype(f32), (0, pad_n),
                   constant_values=99999.0).reshape(N_PAD, 1)
    bcol = jnp.pad(batch.astype(f32), (0, pad_n),
                   constant_values=99999.0).reshape(N_PAD, 1)

    atom_pad = _pad2(params['atom_emb'], DP, DP)
    vn0 = jnp.tile(_pad1(params['vn_emb'], DP)[None, :], (G, 1))
    wp_pad = _pad2(params['wp'], DP, params['wp'].shape[1])
    bp = params['bp'][None, :]

    layer_w = []
    for lp in params['layers']:
        w1 = _pad2(lp['w1'] * lp['g1'][None, :], DP, 2 * D)
        c1 = (lp['b1'] * lp['g1'] + lp['be1'])[None, :]
        w2 = _pad2(lp['w2'] * lp['gn'][None, :], 2 * D, DP)
        c2 = _pad1(lp['b2'] * lp['gn'] + lp['bn'], DP)[None, :]
        bond = _pad2(lp['bond_emb'], BV, DP)
        layer_w.append((w1, c1, w2, c2, bond))
    vn_w = []
    for vp in params['vn_mlps']:
        w1 = _pad2(vp['w1'] * vp['g1'][None, :], DP, 2 * D)
        c1 = (vp['b1'] * vp['g1'] + vp['be1'])[None, :]
        w2 = _pad2(vp['w2'] * vp['g2'][None, :], 2 * D, DP)
        c2 = _pad1(vp['b2'] * vp['g2'] + vp['be2'], DP)[None, :]
        vn_w.append((w1, c1, w2, c2))

    zeros_hbm = jnp.zeros((AGG_ROWS, DP), f32)

    _sc_mesh = plsc.VectorSubcoreMesh(core_axis_name="c", subcore_axis_name="s")
    sc_params = pltpu.CompilerParams(needs_layout_passes=False)
    bucketize = functools.partial(
        pl.kernel,
        mesh=_sc_mesh,
        compiler_params=sc_params,
        out_type=[
            jax.ShapeDtypeStruct((NCHUNK * NW * CAP,), i32),
            jax.ShapeDtypeStruct((NCHUNK * NW * CAP,), i32),
            jax.ShapeDtypeStruct((NW * 16,), i32),
        ],
        scratch_types=(
            [pltpu.VMEM((EB,), i32) for _ in range(3)]
            + [pltpu.VMEM((STG,), i32) for _ in range(8)]
            + [pltpu.VMEM((16,), i32)]
        ),
    )(_bucketize_body)
    gidx_b, dl_b, cnt_b = bucketize(src_p, dst_p, attr_p)

    edge_pass = functools.partial(
        pl.kernel,
        mesh=_sc_mesh,
        compiler_params=sc_params,
        out_type=jax.ShapeDtypeStruct((N_PAD, DP), f32),
        scratch_types=[
            pltpu.VMEM((KB,), i32),
            pltpu.VMEM((KB,), i32),
            pltpu.VMEM((KB, DP), f32),
            pltpu.VMEM((NW * 16,), i32),
            pltpu.VMEM_SHARED((AGG_ROWS, DP), f32),
            pltpu.SemaphoreType.DMA,
        ],
    )(_edge_body)

    vspec = pl.BlockSpec((BN, DP), lambda i: (i, 0))
    cspec = pl.BlockSpec((BN, 1), lambda i: (i, 0))
    gspec = pl.BlockSpec((G, DP), lambda i: (0, 0))
    g1spec = pl.BlockSpec((G, 1), lambda i: (0, 0))

    def full(shape):
        return pl.BlockSpec(shape, lambda *a: tuple(0 for _ in shape))

    h0, ss_h, cnt_g = pl.pallas_call(
        _embed_body,
        grid=(NB,),
        in_specs=[cspec, cspec, full((DP, DP))],
        out_specs=[vspec, gspec, g1spec],
        out_shape=[
            jax.ShapeDtypeStruct((N_PAD, DP), f32),
            jax.ShapeDtypeStruct((G, DP), f32),
            jax.ShapeDtypeStruct((G, 1), f32),
        ],
    )(xcol, bcol, atom_pad)

    hl_call = pl.pallas_call(
        _hl_body,
        grid=(NB,),
        in_specs=[vspec, cspec, gspec],
        out_specs=vspec,
        out_shape=jax.ShapeDtypeStruct((N_PAD, DP), f32),
    )

    tt_call = pl.pallas_call(
        _tt_body,
        grid=(BV, NB),
        in_specs=[pl.BlockSpec((BN, DP), lambda a, i: (i, 0)),
                  pl.BlockSpec((1, 1, DP), lambda a, i: (a, 0, 0))],
        out_specs=pl.BlockSpec((1, BN, DP), lambda a, i: (a, i, 0)),
        out_shape=jax.ShapeDtypeStruct((BV, N_PAD, DP), f32),
    )

    def post_call(last):
        return pl.pallas_call(
            functools.partial(_post_body, last=last),
            grid=(NB,),
            in_specs=[vspec, vspec, cspec, full((DP, 2 * D)), full((1, 2 * D)),
                      full((2 * D, DP)), full((1, DP))],
            out_specs=[vspec, gspec],
            out_shape=[
                jax.ShapeDtypeStruct((N_PAD, DP), f32),
                jax.ShapeDtypeStruct((G, DP), f32),
            ],
        )

    vn_call = pl.pallas_call(
        _vn_body,
        in_specs=[full((G, DP)), full((G, 1)), full((G, DP)),
                  full((DP, 2 * D)), full((1, 2 * D)),
                  full((2 * D, DP)), full((1, DP))],
        out_specs=full((G, DP)),
        out_shape=jax.ShapeDtypeStruct((G, DP), f32),
    )

    head_call = pl.pallas_call(
        _head_body,
        in_specs=[full((G, DP)), full((G, 1)),
                  full((DP, wp_pad.shape[1])), full((1, wp_pad.shape[1]))],
        out_specs=full((G, wp_pad.shape[1])),
        out_shape=jax.ShapeDtypeStruct((G, wp_pad.shape[1]), f32),
    )

    h = h0
    vn = vn0
    for l in range(NL):
        w1, c1, w2, c2, bond = layer_w[l]
        hl = hl_call(h, bcol, vn)
        t_tab = tt_call(hl, bond.reshape(BV, 1, DP)).reshape(TROWS, DP)
        agg = edge_pass(t_tab, gidx_b, dl_b, cnt_b, zeros_hbm)
        h, ss_next = post_call(l == NL - 1)(hl, agg, bcol, w1, c1, w2, c2)
        if l < NL - 1:
            vw1, vc1, vw2, vc2 = vn_w[l]
            vn = vn_call(ss_h, cnt_g, vn, vw1, vc1, vw2, vc2)
        ss_h = ss_next

    return head_call(ss_h, cnt_g, wp_pad, bp)


# speedup vs baseline: 6.1657x; 1.4820x over previous
import functools

import jax
import jax.numpy as jnp
from jax import lax
from jax.experimental import pallas as pl
from jax.experimental.pallas import tpu as pltpu
from jax.experimental.pallas import tpu_sc as plsc

N = 50000
E = 800000
D = 100
DP = 128
G = 512
NL = 5
AV = 119
BV = 5

BN = 1024
NB = 49
BNT = 7168
NBT = 7
N_PAD = NB * BN
NCHUNK = 4
CHUNK = N_PAD // NCHUNK
AGG_ROWS = 12672
TRASH0 = CHUNK
NW = 32
SHARE = 25088
E_PAD = NW * SHARE
CAP = SHARE
TROWS = BV * N_PAD
OB = 1024
SB = 64
EB = 4096
STG = 4352

_HI = lax.Precision.HIGHEST


def _bucketize_body(src_ref, dst_ref, attr_ref, gidx_ref, dl_ref, cnt_ref,
                    sbuf, dbuf, abuf,
                    stg0, std0, stg1, std1, stg2, std2, stg3, std3, cbuf):
    cidx = lax.axis_index("c")
    sidx = lax.axis_index("s")
    wid = sidx * 2 + cidx
    base = wid * SHARE
    lanes = lax.broadcasted_iota(jnp.int32, (16,), 0)
    stgs = ((stg0, std0), (stg1, std1), (stg2, std2), (stg3, std3))
    offs = [jnp.int32(0)] * NCHUNK
    fills = [jnp.int32(0)] * NCHUNK

    block_sizes = [EB] * (SHARE // EB)
    if SHARE % EB:
        block_sizes.append(SHARE % EB)
    boff = 0
    for bs in block_sizes:
        pltpu.sync_copy(src_ref.at[pl.ds(pl.multiple_of(base + boff, 8), bs)], sbuf.at[pl.ds(0, bs)])
        pltpu.sync_copy(dst_ref.at[pl.ds(pl.multiple_of(base + boff, 8), bs)], dbuf.at[pl.ds(0, bs)])
        pltpu.sync_copy(attr_ref.at[pl.ds(pl.multiple_of(base + boff, 8), bs)], abuf.at[pl.ds(0, bs)])

        def vbody(v, carry, boff=boff):
            sl = pl.ds(v * 16, 16)
            s = sbuf[sl]
            d = dbuf[sl]
            a = abuf[sl]
            pos = base + boff + v * 16 + lanes
            valid = pos < E
            c = ((d >= CHUNK).astype(jnp.int32)
                 + (d >= 2 * CHUNK).astype(jnp.int32)
                 + (d >= 3 * CHUNK).astype(jnp.int32))
            dl = d - c * CHUNK
            gx = a * N_PAD + s
            new = []
            for b in range(NCHUNK):
                ob = carry[b]
                m = jnp.logical_and(valid, c == b)
                pos = ob + plsc.cumsum(m.astype(jnp.int32)) - 1
                plsc.store_scatter(stgs[b][0], [pos], gx, mask=m)
                plsc.store_scatter(stgs[b][1], [pos], dl, mask=m)
                cnt = jnp.max(plsc.all_reduce_population_count(m))
                new.append(ob + cnt)
            return tuple(new)

        offs = list(lax.fori_loop(0, bs // 16, vbody, tuple(offs)))
        boff += bs

        for b in range(NCHUNK):
            sg, sd = stgs[b]
            rbase = (b * NW + wid) * CAP
            fill0 = fills[b]
            nfl = offs[b] >> 10

            def fl(j, _, sg=sg, sd=sd, rbase=rbase, fill0=fill0):
                pltpu.sync_copy(sg.at[pl.ds(j * 1024, 1024)],
                                gidx_ref.at[pl.ds(pl.multiple_of(rbase + fill0 + j * 1024, 8), 1024)])
                pltpu.sync_copy(sd.at[pl.ds(j * 1024, 1024)],
                                dl_ref.at[pl.ds(pl.multiple_of(rbase + fill0 + j * 1024, 8), 1024)])
                return 0

            lax.fori_loop(0, nfl, fl, 0)
            flushed = nfl << 10
            newoff = offs[b] - flushed
            nsh = (newoff + 15) >> 4

            def sh(j, _, sg=sg, sd=sd, flushed=flushed):
                sg[pl.ds(pl.multiple_of(j * 16, 8), 16)] = sg[pl.ds(pl.multiple_of(flushed + j * 16, 8), 16)]
                sd[pl.ds(pl.multiple_of(j * 16, 8), 16)] = sd[pl.ds(pl.multiple_of(flushed + j * 16, 8), 16)]
                return 0

            lax.fori_loop(0, nsh, sh, 0)
            fills[b] = fill0 + flushed
            offs[b] = newoff

    counts_vec = jnp.zeros((16,), jnp.int32)
    for b in range(NCHUNK):
        sg, sd = stgs[b]
        rbase = (b * NW + wid) * CAP
        fill0 = fills[b]
        total = fill0 + offs[b]
        nfl = (offs[b] + 15) >> 4

        def tf(j, _, sg=sg, sd=sd, rbase=rbase, fill0=fill0):
            pltpu.sync_copy(sg.at[pl.ds(pl.multiple_of(j * 16, 8), 16)],
                            gidx_ref.at[pl.ds(pl.multiple_of(rbase + fill0 + j * 16, 8), 16)])
            pltpu.sync_copy(sd.at[pl.ds(pl.multiple_of(j * 16, 8), 16)],
                            dl_ref.at[pl.ds(pl.multiple_of(rbase + fill0 + j * 16, 8), 16)])
            return 0

        lax.fori_loop(0, nfl, tf, 0)
        counts_vec = counts_vec + jnp.where(lanes == b, total, 0)
    cbuf[...] = counts_vec
    pltpu.sync_copy(cbuf, cnt_ref.at[pl.ds(pl.multiple_of(wid * 16, 8), 16)])


def _edge_body(t_ref, gidx_ref, dl_ref, cnt_ref, zeros_ref, agg_ref,
               gixb, dlb, dli2, rows0, rows1, cnts, aggS, sem):
    cidx = lax.axis_index("c")
    sidx = lax.axis_index("s")
    lanes = lax.broadcasted_iota(jnp.int32, (16,), 0)
    rowbufs = (rows0, rows1)
    pltpu.sync_copy(cnt_ref, cnts)
    for p in range(2):
        c = cidx * 2 + p
        pltpu.sync_copy(zeros_ref.at[pl.ds(pl.multiple_of(sidx * (AGG_ROWS // 16), 8), AGG_ROWS // 16)],
                        aggS.at[pl.ds(pl.multiple_of(sidx * (AGG_ROWS // 16), 8), AGG_ROWS // 16)])
        plsc.subcore_barrier()
        for wsub in range(2):
            w = sidx * 2 + wsub
            cvec = cnts[pl.ds(pl.multiple_of(w * 16, 8), 16)]
            n = jnp.max(jnp.where(lanes == c, cvec, 0))
            rbase = (c * NW + w) * CAP
            nob = n >> 10

            def obody(j, _, rbase=rbase, n=n):
                obase = j * OB
                pltpu.sync_copy(gidx_ref.at[pl.ds(pl.multiple_of(rbase + obase, 8), OB)], gixb)
                pltpu.sync_copy(dl_ref.at[pl.ds(pl.multiple_of(rbase + obase, 8), OB)], dlb)
                for v in range(OB // 16):
                    sl = pl.ds(v * 16, 16)
                    posv = obase + v * 16 + lanes
                    mv = posv < n
                    gixb[sl] = jnp.where(mv, gixb[sl], posv & 2047)
                    dli2[v // (SB // 16), pl.ds((v % (SB // 16)) * 16, 16)] = jnp.where(
                        mv, dlb[sl], TRASH0 + (posv & 127))
                hprev = None
                sbprev = -1
                for sb in range(OB // SB):
                    h = pltpu.async_copy(
                        t_ref.at[gixb.at[pl.ds(pl.multiple_of(sb * SB, 8), SB)]],
                        rowbufs[sb & 1], sem)
                    if hprev is not None:
                        hprev.wait()
                        pltpu.sync_copy(rowbufs[sbprev & 1], aggS.at[dli2.at[sbprev]],
                                        add=True)
                    hprev, sbprev = h, sb
                hprev.wait()
                pltpu.sync_copy(rowbufs[sbprev & 1], aggS.at[dli2.at[sbprev]], add=True)
                return 0

            lax.fori_loop(0, nob, obody, 0)
            ntail = (n - (nob << 10) + SB - 1) >> 6

            def tbody(t, _, rbase=rbase, n=n, nob=nob):
                boff = (nob << 10) + t * SB
                pltpu.sync_copy(gidx_ref.at[pl.ds(pl.multiple_of(rbase + boff, 8), SB)],
                                gixb.at[pl.ds(0, SB)])
                pltpu.sync_copy(dl_ref.at[pl.ds(pl.multiple_of(rbase + boff, 8), SB)],
                                dlb.at[pl.ds(0, SB)])
                for v in range(SB // 16):
                    sl = pl.ds(v * 16, 16)
                    posv = boff + v * 16 + lanes
                    mv = posv < n
                    gixb[sl] = jnp.where(mv, gixb[sl], posv & 2047)
                    dli2[0, pl.ds(v * 16, 16)] = jnp.where(
                        mv, dlb[sl], TRASH0 + (posv & 127))
                pltpu.async_copy(t_ref.at[gixb.at[pl.ds(0, SB)]], rows0, sem).wait()
                pltpu.sync_copy(rows0, aggS.at[dli2.at[0]], add=True)
                return 0

            lax.fori_loop(0, ntail, tbody, 0)
        plsc.subcore_barrier()
        pltpu.sync_copy(aggS.at[pl.ds(pl.multiple_of(sidx * (CHUNK // 16), 8), CHUNK // 16)],
                        agg_ref.at[pl.ds(pl.multiple_of(c * CHUNK + sidx * (CHUNK // 16), 8), CHUNK // 16)])
        plsc.subcore_barrier()


def _embed_body(xcol_ref, bcol_ref, emb_ref, h_ref, ss_ref, cnt_ref):
    ohx = (xcol_ref[...] == lax.broadcasted_iota(jnp.int32, (BN, DP), 1).astype(jnp.float32))
    ohx = ohx.astype(jnp.float32)
    h = jnp.dot(ohx, emb_ref[...], preferred_element_type=jnp.float32,
                precision=_HI)
    h_ref[...] = h
    ohb = (bcol_ref[...] == lax.broadcasted_iota(jnp.int32, (BN, G), 1).astype(jnp.float32))
    ohb = ohb.astype(jnp.float32)
    ss = lax.dot_general(ohb, h, (((0,), (0,)), ((), ())),
                         preferred_element_type=jnp.float32, precision=_HI)
    cnt = lax.dot_general(ohb, jnp.ones((BN, 1), jnp.float32),
                          (((0,), (0,)), ((), ())),
                          preferred_element_type=jnp.float32, precision=_HI)

    @pl.when(pl.program_id(0) == 0)
    def _():
        ss_ref[...] = ss
        cnt_ref[...] = cnt

    @pl.when(pl.program_id(0) != 0)
    def _():
        ss_ref[...] += ss
        cnt_ref[...] += cnt


def _hl_body(h_ref, bcol_ref, vn_ref, hl_ref):
    ohb = (bcol_ref[...] == lax.broadcasted_iota(jnp.int32, (BN, G), 1).astype(jnp.float32))
    ohb = ohb.astype(jnp.float32)
    hl_ref[...] = h_ref[...] + jnp.dot(ohb, vn_ref[...],
                                       preferred_element_type=jnp.float32,
                                       precision=_HI)


def _tt_body(hl_ref, bond_ref, t_ref):
    t_ref[0] = jnp.maximum(hl_ref[...] + bond_ref[0], 0.0)


def _post_body(hl_ref, agg_ref, bcol_ref, w1_ref, c1_ref, w2_ref, c2_ref,
               h_ref, ss_ref, *, last):
    z = hl_ref[...] + agg_ref[...]
    z1 = jnp.maximum(jnp.dot(z, w1_ref[...],
                             preferred_element_type=jnp.float32,
                             precision=_HI) + c1_ref[...], 0.0)
    h2 = jnp.dot(z1, w2_ref[...], preferred_element_type=jnp.float32,
                 precision=_HI) + c2_ref[...]
    if not last:
        h2 = jnp.maximum(h2, 0.0)
    h_ref[...] = h2
    ohb = (bcol_ref[...] == lax.broadcasted_iota(jnp.int32, (BN, G), 1).astype(jnp.float32))
    ohb = ohb.astype(jnp.float32)
    ss = lax.dot_general(ohb, h2, (((0,), (0,)), ((), ())),
                         preferred_element_type=jnp.float32, precision=_HI)

    @pl.when(pl.program_id(0) == 0)
    def _():
        ss_ref[...] = ss

    @pl.when(pl.program_id(0) != 0)
    def _():
        ss_ref[...] += ss


def _vn_body(ss_ref, cnt_ref, vn_ref, w1_ref, c1_ref, w2_ref, c2_ref, out_ref):
    vt = ss_ref[...] + (cnt_ref[...] + 1.0) * vn_ref[...]
    t = jnp.maximum(jnp.dot(vt, w1_ref[...],
                            preferred_element_type=jnp.float32,
                            precision=_HI) + c1_ref[...], 0.0)
    out_ref[...] = jnp.maximum(jnp.dot(t, w2_ref[...],
                                       preferred_element_type=jnp.float32,
                                       precision=_HI) + c2_ref[...], 0.0)


def _head_body(ss_ref, cnt_ref, wp_ref, bp_ref, out_ref):
    hg = ss_ref[...] / jnp.maximum(cnt_ref[...], 1.0)
    out_ref[...] = jnp.dot(hg, wp_ref[...], preferred_element_type=jnp.float32,
                           precision=_HI) + bp_ref[...]


def _pad2(w, rows, cols):
    out = jnp.zeros((rows, cols), jnp.float32)
    return out.at[:w.shape[0], :w.shape[1]].set(w)


def _pad1(v, n):
    out = jnp.zeros((n,), jnp.float32)
    return out.at[:v.shape[0]].set(v)


def kernel(x, edge_index, edge_attr, batch, params):
    f32 = jnp.float32
    i32 = jnp.int32

    src = edge_index[0].astype(i32)
    dst = edge_index[1].astype(i32)
    attr = edge_attr.astype(i32)
    pad_e = E_PAD - E
    src_p = jnp.pad(src, (0, pad_e))
    dst_p = jnp.pad(dst, (0, pad_e))
    attr_p = jnp.pad(attr, (0, pad_e))

    pad_n = N_PAD - N
    xcol = jnp.pad(x.astype(f32), (0, pad_n),
                   constant_values=99999.0).reshape(N_PAD, 1)
    bcol = jnp.pad(batch.astype(f32), (0, pad_n),
                   constant_values=99999.0).reshape(N_PAD, 1)

    atom_pad = _pad2(params['atom_emb'], DP, DP)
    vn0 = jnp.tile(_pad1(params['vn_emb'], DP)[None, :], (G, 1))
    wp_pad = _pad2(params['wp'], DP, params['wp'].shape[1])
    bp = params['bp'][None, :]

    layer_w = []
    for lp in params['layers']:
        w1 = _pad2(lp['w1'] * lp['g1'][None, :], DP, 2 * D)
        c1 = (lp['b1'] * lp['g1'] + lp['be1'])[None, :]
        w2 = _pad2(lp['w2'] * lp['gn'][None, :], 2 * D, DP)
        c2 = _pad1(lp['b2'] * lp['gn'] + lp['bn'], DP)[None, :]
        bond = _pad2(lp['bond_emb'], BV, DP)
        layer_w.append((w1, c1, w2, c2, bond))
    vn_w = []
    for vp in params['vn_mlps']:
        w1 = _pad2(vp['w1'] * vp['g1'][None, :], DP, 2 * D)
        c1 = (vp['b1'] * vp['g1'] + vp['be1'])[None, :]
        w2 = _pad2(vp['w2'] * vp['g2'][None, :], 2 * D, DP)
        c2 = _pad1(vp['b2'] * vp['g2'] + vp['be2'], DP)[None, :]
        vn_w.append((w1, c1, w2, c2))

    zeros_hbm = jnp.zeros((AGG_ROWS, DP), f32)

    _sc_mesh = plsc.VectorSubcoreMesh(core_axis_name="c", subcore_axis_name="s")
    sc_params = pltpu.CompilerParams(needs_layout_passes=False)
    bucketize = functools.partial(
        pl.kernel,
        mesh=_sc_mesh,
        compiler_params=sc_params,
        out_type=[
            jax.ShapeDtypeStruct((NCHUNK * NW * CAP,), i32),
            jax.ShapeDtypeStruct((NCHUNK * NW * CAP,), i32),
            jax.ShapeDtypeStruct((NW * 16,), i32),
        ],
        scratch_types=(
            [pltpu.VMEM((EB,), i32) for _ in range(3)]
            + [pltpu.VMEM((STG,), i32) for _ in range(8)]
            + [pltpu.VMEM((16,), i32)]
        ),
    )(_bucketize_body)
    gidx_b, dl_b, cnt_b = bucketize(src_p, dst_p, attr_p)

    edge_pass = functools.partial(
        pl.kernel,
        mesh=_sc_mesh,
        compiler_params=sc_params,
        out_type=jax.ShapeDtypeStruct((N_PAD, DP), f32),
        scratch_types=[
            pltpu.VMEM((OB,), i32),
            pltpu.VMEM((OB,), i32),
            pltpu.VMEM((OB // SB, SB), i32),
            pltpu.VMEM((SB, DP), f32),
            pltpu.VMEM((SB, DP), f32),
            pltpu.VMEM((NW * 16,), i32),
            pltpu.VMEM_SHARED((AGG_ROWS, DP), f32),
            pltpu.SemaphoreType.DMA,
        ],
    )(_edge_body)

    vspec = pl.BlockSpec((BN, DP), lambda i: (i, 0))
    cspec = pl.BlockSpec((BN, 1), lambda i: (i, 0))
    gspec = pl.BlockSpec((G, DP), lambda i: (0, 0))
    g1spec = pl.BlockSpec((G, 1), lambda i: (0, 0))

    def full(shape):
        return pl.BlockSpec(shape, lambda *a: tuple(0 for _ in shape))

    h0, ss_h, cnt_g = pl.pallas_call(
        _embed_body,
        grid=(NB,),
        in_specs=[cspec, cspec, full((DP, DP))],
        out_specs=[vspec, gspec, g1spec],
        out_shape=[
            jax.ShapeDtypeStruct((N_PAD, DP), f32),
            jax.ShapeDtypeStruct((G, DP), f32),
            jax.ShapeDtypeStruct((G, 1), f32),
        ],
    )(xcol, bcol, atom_pad)

    hl_call = pl.pallas_call(
        _hl_body,
        grid=(NB,),
        in_specs=[vspec, cspec, gspec],
        out_specs=vspec,
        out_shape=jax.ShapeDtypeStruct((N_PAD, DP), f32),
    )

    tt_call = pl.pallas_call(
        _tt_body,
        grid=(BV, NBT),
        in_specs=[pl.BlockSpec((BNT, DP), lambda a, i: (i, 0)),
                  pl.BlockSpec((1, 1, DP), lambda a, i: (a, 0, 0))],
        out_specs=pl.BlockSpec((1, BNT, DP), lambda a, i: (a, i, 0)),
        out_shape=jax.ShapeDtypeStruct((BV, N_PAD, DP), f32),
    )

    def post_call(last):
        return pl.pallas_call(
            functools.partial(_post_body, last=last),
            grid=(NB,),
            in_specs=[vspec, vspec, cspec, full((DP, 2 * D)), full((1, 2 * D)),
                      full((2 * D, DP)), full((1, DP))],
            out_specs=[vspec, gspec],
            out_shape=[
                jax.ShapeDtypeStruct((N_PAD, DP), f32),
                jax.ShapeDtypeStruct((G, DP), f32),
            ],
        )

    vn_call = pl.pallas_call(
        _vn_body,
        in_specs=[full((G, DP)), full((G, 1)), full((G, DP)),
                  full((DP, 2 * D)), full((1, 2 * D)),
                  full((2 * D, DP)), full((1, DP))],
        out_specs=full((G, DP)),
        out_shape=jax.ShapeDtypeStruct((G, DP), f32),
    )

    head_call = pl.pallas_call(
        _head_body,
        in_specs=[full((G, DP)), full((G, 1)),
                  full((DP, wp_pad.shape[1])), full((1, wp_pad.shape[1]))],
        out_specs=full((G, wp_pad.shape[1])),
        out_shape=jax.ShapeDtypeStruct((G, wp_pad.shape[1]), f32),
    )

    h = h0
    vn = vn0
    for l in range(NL):
        w1, c1, w2, c2, bond = layer_w[l]
        hl = hl_call(h, bcol, vn)
        t_tab = tt_call(hl, bond.reshape(BV, 1, DP)).reshape(TROWS, DP)
        agg = edge_pass(t_tab, gidx_b, dl_b, cnt_b, zeros_hbm)
        h, ss_next = post_call(l == NL - 1)(hl, agg, bcol, w1, c1, w2, c2)
        if l < NL - 1:
            vw1, vc1, vw2, vc2 = vn_w[l]
            vn = vn_call(ss_h, cnt_g, vn, vw1, vc1, vw2, vc2)
        ss_h = ss_next

    return head_call(ss_h, cnt_g, wp_pad, bp)


# speedup vs baseline: 6.2272x; 1.0100x over previous
import functools

import jax
import jax.numpy as jnp
from jax import lax
from jax.experimental import pallas as pl
from jax.experimental.pallas import tpu as pltpu
from jax.experimental.pallas import tpu_sc as plsc

N = 50000
E = 800000
D = 100
DP = 128
G = 512
NL = 5
AV = 119
BV = 5

BN = 1024
NB = 49
BNT = 7168
NBT = 7
N_PAD = NB * BN
NCHUNK = 4
CHUNK = N_PAD // NCHUNK
AGG_ROWS = 12672
TRASH0 = CHUNK
NW = 32
SHARE = 25088
E_PAD = NW * SHARE
CAP = SHARE
TROWS = BV * N_PAD
OB = 1024
SB = 64
EB = 4096
STG = 4352

_HI = lax.Precision.HIGHEST


def _bucketize_body(src_ref, dst_ref, attr_ref, gidx_ref, dl_ref, cnt_ref,
                    sbuf, dbuf, abuf,
                    stg0, std0, stg1, std1, stg2, std2, stg3, std3, cbuf):
    cidx = lax.axis_index("c")
    sidx = lax.axis_index("s")
    wid = sidx * 2 + cidx
    base = wid * SHARE
    lanes = lax.broadcasted_iota(jnp.int32, (16,), 0)
    stgs = ((stg0, std0), (stg1, std1), (stg2, std2), (stg3, std3))
    offs = [jnp.int32(0)] * NCHUNK
    fills = [jnp.int32(0)] * NCHUNK

    block_sizes = [EB] * (SHARE // EB)
    if SHARE % EB:
        block_sizes.append(SHARE % EB)
    boff = 0
    for bs in block_sizes:
        pltpu.sync_copy(src_ref.at[pl.ds(pl.multiple_of(base + boff, 8), bs)], sbuf.at[pl.ds(0, bs)])
        pltpu.sync_copy(dst_ref.at[pl.ds(pl.multiple_of(base + boff, 8), bs)], dbuf.at[pl.ds(0, bs)])
        pltpu.sync_copy(attr_ref.at[pl.ds(pl.multiple_of(base + boff, 8), bs)], abuf.at[pl.ds(0, bs)])

        def vbody(v, carry, boff=boff):
            sl = pl.ds(v * 16, 16)
            s = sbuf[sl]
            d = dbuf[sl]
            a = abuf[sl]
            pos = base + boff + v * 16 + lanes
            valid = pos < E
            c = ((d >= CHUNK).astype(jnp.int32)
                 + (d >= 2 * CHUNK).astype(jnp.int32)
                 + (d >= 3 * CHUNK).astype(jnp.int32))
            dl = d - c * CHUNK
            gx = a * N_PAD + s
            new = []
            for b in range(NCHUNK):
                ob = carry[b]
                m = jnp.logical_and(valid, c == b)
                pos = ob + plsc.cumsum(m.astype(jnp.int32)) - 1
                plsc.store_scatter(stgs[b][0], [pos], gx, mask=m)
                plsc.store_scatter(stgs[b][1], [pos], dl, mask=m)
                cnt = jnp.max(plsc.all_reduce_population_count(m))
                new.append(ob + cnt)
            return tuple(new)

        offs = list(lax.fori_loop(0, bs // 16, vbody, tuple(offs)))
        boff += bs

        for b in range(NCHUNK):
            sg, sd = stgs[b]
            rbase = (b * NW + wid) * CAP
            fill0 = fills[b]
            nfl = offs[b] >> 10

            def fl(j, _, sg=sg, sd=sd, rbase=rbase, fill0=fill0):
                pltpu.sync_copy(sg.at[pl.ds(j * 1024, 1024)],
                                gidx_ref.at[pl.ds(pl.multiple_of(rbase + fill0 + j * 1024, 8), 1024)])
                pltpu.sync_copy(sd.at[pl.ds(j * 1024, 1024)],
                                dl_ref.at[pl.ds(pl.multiple_of(rbase + fill0 + j * 1024, 8), 1024)])
                return 0

            lax.fori_loop(0, nfl, fl, 0)
            flushed = nfl << 10
            newoff = offs[b] - flushed
            nsh = (newoff + 15) >> 4

            def sh(j, _, sg=sg, sd=sd, flushed=flushed):
                sg[pl.ds(pl.multiple_of(j * 16, 8), 16)] = sg[pl.ds(pl.multiple_of(flushed + j * 16, 8), 16)]
                sd[pl.ds(pl.multiple_of(j * 16, 8), 16)] = sd[pl.ds(pl.multiple_of(flushed + j * 16, 8), 16)]
                return 0

            lax.fori_loop(0, nsh, sh, 0)
            fills[b] = fill0 + flushed
            offs[b] = newoff

    counts_vec = jnp.zeros((16,), jnp.int32)
    for b in range(NCHUNK):
        sg, sd = stgs[b]
        rbase = (b * NW + wid) * CAP
        fill0 = fills[b]
        total = fill0 + offs[b]
        nfl = (offs[b] + 15) >> 4

        def tf(j, _, sg=sg, sd=sd, rbase=rbase, fill0=fill0):
            pltpu.sync_copy(sg.at[pl.ds(pl.multiple_of(j * 16, 8), 16)],
                            gidx_ref.at[pl.ds(pl.multiple_of(rbase + fill0 + j * 16, 8), 16)])
            pltpu.sync_copy(sd.at[pl.ds(pl.multiple_of(j * 16, 8), 16)],
                            dl_ref.at[pl.ds(pl.multiple_of(rbase + fill0 + j * 16, 8), 16)])
            return 0

        lax.fori_loop(0, nfl, tf, 0)
        counts_vec = counts_vec + jnp.where(lanes == b, total, 0)
    cbuf[...] = counts_vec
    pltpu.sync_copy(cbuf, cnt_ref.at[pl.ds(pl.multiple_of(wid * 16, 8), 16)])


def _edge_body(t_ref, gidx_ref, dl_ref, cnt_ref, zeros_ref, agg_ref,
               gixb, dlb, dli2, rows0, rows1, cnts, aggS, sem, sems):
    cidx = lax.axis_index("c")
    sidx = lax.axis_index("s")
    lanes = lax.broadcasted_iota(jnp.int32, (16,), 0)
    rowbufs = (rows0, rows1)
    pltpu.sync_copy(cnt_ref, cnts)
    for p in range(2):
        c = cidx * 2 + p
        pltpu.sync_copy(zeros_ref.at[pl.ds(pl.multiple_of(sidx * (AGG_ROWS // 16), 8), AGG_ROWS // 16)],
                        aggS.at[pl.ds(pl.multiple_of(sidx * (AGG_ROWS // 16), 8), AGG_ROWS // 16)])
        plsc.subcore_barrier()
        for wsub in range(2):
            w = sidx * 2 + wsub
            cvec = cnts[pl.ds(pl.multiple_of(w * 16, 8), 16)]
            n = jnp.max(jnp.where(lanes == c, cvec, 0))
            rbase = (c * NW + w) * CAP
            nob = n >> 10

            def obody(j, _, rbase=rbase, n=n):
                obase = j * OB
                pltpu.sync_copy(gidx_ref.at[pl.ds(pl.multiple_of(rbase + obase, 8), OB)], gixb)
                pltpu.sync_copy(dl_ref.at[pl.ds(pl.multiple_of(rbase + obase, 8), OB)], dlb)
                for v in range(OB // 16):
                    sl = pl.ds(v * 16, 16)
                    posv = obase + v * 16 + lanes
                    mv = posv < n
                    gixb[sl] = jnp.where(mv, gixb[sl], posv & 2047)
                    dli2[v // (SB // 16), pl.ds((v % (SB // 16)) * 16, 16)] = jnp.where(
                        mv, dlb[sl], TRASH0 + (posv & 127))
                ns = OB // SB
                hg = [None] * ns
                hs = [None] * ns
                for sb in range(ns):
                    if sb >= 2:
                        hs[sb - 2].wait()
                    hg[sb] = pltpu.async_copy(
                        t_ref.at[gixb.at[pl.ds(pl.multiple_of(sb * SB, 8), SB)]],
                        rowbufs[sb & 1], sem)
                    if sb >= 1:
                        hg[sb - 1].wait()
                        hs[sb - 1] = pltpu.async_copy(
                            rowbufs[(sb - 1) & 1], aggS.at[dli2.at[sb - 1]], sems,
                            add=True)
                hg[ns - 1].wait()
                hs[ns - 1] = pltpu.async_copy(
                    rowbufs[(ns - 1) & 1], aggS.at[dli2.at[ns - 1]], sems, add=True)
                hs[ns - 2].wait()
                hs[ns - 1].wait()
                return 0

            lax.fori_loop(0, nob, obody, 0)
            ntail = (n - (nob << 10) + SB - 1) >> 6

            def tbody(t, _, rbase=rbase, n=n, nob=nob):
                boff = (nob << 10) + t * SB
                pltpu.sync_copy(gidx_ref.at[pl.ds(pl.multiple_of(rbase + boff, 8), SB)],
                                gixb.at[pl.ds(0, SB)])
                pltpu.sync_copy(dl_ref.at[pl.ds(pl.multiple_of(rbase + boff, 8), SB)],
                                dlb.at[pl.ds(0, SB)])
                for v in range(SB // 16):
                    sl = pl.ds(v * 16, 16)
                    posv = boff + v * 16 + lanes
                    mv = posv < n
                    gixb[sl] = jnp.where(mv, gixb[sl], posv & 2047)
                    dli2[0, pl.ds(v * 16, 16)] = jnp.where(
                        mv, dlb[sl], TRASH0 + (posv & 127))
                pltpu.async_copy(t_ref.at[gixb.at[pl.ds(0, SB)]], rows0, sem).wait()
                pltpu.sync_copy(rows0, aggS.at[dli2.at[0]], add=True)
                return 0

            lax.fori_loop(0, ntail, tbody, 0)
        plsc.subcore_barrier()
        pltpu.sync_copy(aggS.at[pl.ds(pl.multiple_of(sidx * (CHUNK // 16), 8), CHUNK // 16)],
                        agg_ref.at[pl.ds(pl.multiple_of(c * CHUNK + sidx * (CHUNK // 16), 8), CHUNK // 16)])
        plsc.subcore_barrier()


def _embed_body(xcol_ref, brow_ref, emb_ref, h_ref, ss_ref, cnt_ref):
    ohx = (xcol_ref[...] == lax.broadcasted_iota(jnp.int32, (BN, DP), 1).astype(jnp.float32))
    ohx = ohx.astype(jnp.float32)
    h = jnp.dot(ohx, emb_ref[...], preferred_element_type=jnp.float32,
                precision=_HI)
    h_ref[...] = h
    ohbT = (brow_ref[...] == lax.broadcasted_iota(jnp.int32, (G, BN), 0).astype(jnp.float32))
    ohbT = ohbT.astype(jnp.float32)
    ss = jnp.dot(ohbT, h, preferred_element_type=jnp.float32, precision=_HI)
    cnt = jnp.sum(ohbT, axis=1, keepdims=True)

    @pl.when(pl.program_id(0) == 0)
    def _():
        ss_ref[...] = ss
        cnt_ref[...] = cnt

    @pl.when(pl.program_id(0) != 0)
    def _():
        ss_ref[...] += ss
        cnt_ref[...] += cnt


def _hl_body(h_ref, bcol_ref, vn_ref, hl_ref):
    ohb = (bcol_ref[...] == lax.broadcasted_iota(jnp.int32, (BN, G), 1).astype(jnp.float32))
    ohb = ohb.astype(jnp.float32)
    hl_ref[...] = h_ref[...] + jnp.dot(ohb, vn_ref[...],
                                       preferred_element_type=jnp.float32,
                                       precision=_HI)


def _tt_body(hl_ref, bond_ref, t_ref):
    t_ref[0] = jnp.maximum(hl_ref[...] + bond_ref[0], 0.0)


def _post_body(hl_ref, agg_ref, brow_ref, w1_ref, c1_ref, w2_ref, c2_ref,
               h_ref, ss_ref, *, last):
    z = hl_ref[...] + agg_ref[...]
    z1 = jnp.maximum(jnp.dot(z, w1_ref[...],
                             preferred_element_type=jnp.float32,
                             precision=_HI) + c1_ref[...], 0.0)
    h2 = jnp.dot(z1, w2_ref[...], preferred_element_type=jnp.float32,
                 precision=_HI) + c2_ref[...]
    if not last:
        h2 = jnp.maximum(h2, 0.0)
    h_ref[...] = h2
    ohbT = (brow_ref[...] == lax.broadcasted_iota(jnp.int32, (G, BN), 0).astype(jnp.float32))
    ohbT = ohbT.astype(jnp.float32)
    ss = jnp.dot(ohbT, h2, preferred_element_type=jnp.float32, precision=_HI)

    @pl.when(pl.program_id(0) == 0)
    def _():
        ss_ref[...] = ss

    @pl.when(pl.program_id(0) != 0)
    def _():
        ss_ref[...] += ss


def _vn_body(ss_ref, cnt_ref, vn_ref, w1_ref, c1_ref, w2_ref, c2_ref, out_ref):
    vt = ss_ref[...] + (cnt_ref[...] + 1.0) * vn_ref[...]
    t = jnp.maximum(jnp.dot(vt, w1_ref[...],
                            preferred_element_type=jnp.float32,
                            precision=_HI) + c1_ref[...], 0.0)
    out_ref[...] = jnp.maximum(jnp.dot(t, w2_ref[...],
                                       preferred_element_type=jnp.float32,
                                       precision=_HI) + c2_ref[...], 0.0)


def _head_body(ss_ref, cnt_ref, wp_ref, bp_ref, out_ref):
    hg = ss_ref[...] / jnp.maximum(cnt_ref[...], 1.0)
    out_ref[...] = jnp.dot(hg, wp_ref[...], preferred_element_type=jnp.float32,
                           precision=_HI) + bp_ref[...]


def _pad2(w, rows, cols):
    out = jnp.zeros((rows, cols), jnp.float32)
    return out.at[:w.shape[0], :w.shape[1]].set(w)


def _pad1(v, n):
    out = jnp.zeros((n,), jnp.float32)
    return out.at[:v.shape[0]].set(v)


def kernel(x, edge_index, edge_attr, batch, params):
    f32 = jnp.float32
    i32 = jnp.int32

    src = edge_index[0].astype(i32)
    dst = edge_index[1].astype(i32)
    attr = edge_attr.astype(i32)
    pad_e = E_PAD - E
    src_p = jnp.pad(src, (0, pad_e))
    dst_p = jnp.pad(dst, (0, pad_e))
    attr_p = jnp.pad(attr, (0, pad_e))

    pad_n = N_PAD - N
    xcol = jnp.pad(x.astype(f32), (0, pad_n),
                   constant_values=99999.0).reshape(N_PAD, 1)
    bflat = jnp.pad(batch.astype(f32), (0, pad_n), constant_values=99999.0)
    bcol = bflat.reshape(N_PAD, 1)
    brow = bflat.reshape(1, N_PAD)

    atom_pad = _pad2(params['atom_emb'], DP, DP)
    vn0 = jnp.tile(_pad1(params['vn_emb'], DP)[None, :], (G, 1))
    wp_pad = _pad2(params['wp'], DP, params['wp'].shape[1])
    bp = params['bp'][None, :]

    layer_w = []
    for lp in params['layers']:
        w1 = _pad2(lp['w1'] * lp['g1'][None, :], DP, 2 * D)
        c1 = (lp['b1'] * lp['g1'] + lp['be1'])[None, :]
        w2 = _pad2(lp['w2'] * lp['gn'][None, :], 2 * D, DP)
        c2 = _pad1(lp['b2'] * lp['gn'] + lp['bn'], DP)[None, :]
        bond = _pad2(lp['bond_emb'], BV, DP)
        layer_w.append((w1, c1, w2, c2, bond))
    vn_w = []
    for vp in params['vn_mlps']:
        w1 = _pad2(vp['w1'] * vp['g1'][None, :], DP, 2 * D)
        c1 = (vp['b1'] * vp['g1'] + vp['be1'])[None, :]
        w2 = _pad2(vp['w2'] * vp['g2'][None, :], 2 * D, DP)
        c2 = _pad1(vp['b2'] * vp['g2'] + vp['be2'], DP)[None, :]
        vn_w.append((w1, c1, w2, c2))

    zeros_hbm = jnp.zeros((AGG_ROWS, DP), f32)

    _sc_mesh = plsc.VectorSubcoreMesh(core_axis_name="c", subcore_axis_name="s")
    sc_params = pltpu.CompilerParams(needs_layout_passes=False)
    bucketize = functools.partial(
        pl.kernel,
        mesh=_sc_mesh,
        compiler_params=sc_params,
        out_type=[
            jax.ShapeDtypeStruct((NCHUNK * NW * CAP,), i32),
            jax.ShapeDtypeStruct((NCHUNK * NW * CAP,), i32),
            jax.ShapeDtypeStruct((NW * 16,), i32),
        ],
        scratch_types=(
            [pltpu.VMEM((EB,), i32) for _ in range(3)]
            + [pltpu.VMEM((STG,), i32) for _ in range(8)]
            + [pltpu.VMEM((16,), i32)]
        ),
    )(_bucketize_body)
    gidx_b, dl_b, cnt_b = bucketize(src_p, dst_p, attr_p)

    edge_pass = functools.partial(
        pl.kernel,
        mesh=_sc_mesh,
        compiler_params=sc_params,
        out_type=jax.ShapeDtypeStruct((N_PAD, DP), f32),
        scratch_types=[
            pltpu.VMEM((OB,), i32),
            pltpu.VMEM((OB,), i32),
            pltpu.VMEM((OB // SB, SB), i32),
            pltpu.VMEM((SB, DP), f32),
            pltpu.VMEM((SB, DP), f32),
            pltpu.VMEM((NW * 16,), i32),
            pltpu.VMEM_SHARED((AGG_ROWS, DP), f32),
            pltpu.SemaphoreType.DMA,
            pltpu.SemaphoreType.DMA,
        ],
    )(_edge_body)

    vspec = pl.BlockSpec((BN, DP), lambda i: (i, 0))
    cspec = pl.BlockSpec((BN, 1), lambda i: (i, 0))
    rspec = pl.BlockSpec((1, BN), lambda i: (0, i))
    gspec = pl.BlockSpec((G, DP), lambda i: (0, 0))
    g1spec = pl.BlockSpec((G, 1), lambda i: (0, 0))

    def full(shape):
        return pl.BlockSpec(shape, lambda *a: tuple(0 for _ in shape))

    h0, ss_h, cnt_g = pl.pallas_call(
        _embed_body,
        grid=(NB,),
        in_specs=[cspec, rspec, full((DP, DP))],
        out_specs=[vspec, gspec, g1spec],
        out_shape=[
            jax.ShapeDtypeStruct((N_PAD, DP), f32),
            jax.ShapeDtypeStruct((G, DP), f32),
            jax.ShapeDtypeStruct((G, 1), f32),
        ],
    )(xcol, brow, atom_pad)

    hl_call = pl.pallas_call(
        _hl_body,
        grid=(NB,),
        in_specs=[vspec, cspec, gspec],
        out_specs=vspec,
        out_shape=jax.ShapeDtypeStruct((N_PAD, DP), f32),
    )

    tt_call = pl.pallas_call(
        _tt_body,
        grid=(BV, NBT),
        in_specs=[pl.BlockSpec((BNT, DP), lambda a, i: (i, 0)),
                  pl.BlockSpec((1, 1, DP), lambda a, i: (a, 0, 0))],
        out_specs=pl.BlockSpec((1, BNT, DP), lambda a, i: (a, i, 0)),
        out_shape=jax.ShapeDtypeStruct((BV, N_PAD, DP), f32),
    )

    def post_call(last):
        return pl.pallas_call(
            functools.partial(_post_body, last=last),
            grid=(NB,),
            in_specs=[vspec, vspec, rspec, full((DP, 2 * D)), full((1, 2 * D)),
                      full((2 * D, DP)), full((1, DP))],
            out_specs=[vspec, gspec],
            out_shape=[
                jax.ShapeDtypeStruct((N_PAD, DP), f32),
                jax.ShapeDtypeStruct((G, DP), f32),
            ],
        )

    vn_call = pl.pallas_call(
        _vn_body,
        in_specs=[full((G, DP)), full((G, 1)), full((G, DP)),
                  full((DP, 2 * D)), full((1, 2 * D)),
                  full((2 * D, DP)), full((1, DP))],
        out_specs=full((G, DP)),
        out_shape=jax.ShapeDtypeStruct((G, DP), f32),
    )

    head_call = pl.pallas_call(
        _head_body,
        in_specs=[full((G, DP)), full((G, 1)),
                  full((DP, wp_pad.shape[1])), full((1, wp_pad.shape[1]))],
        out_specs=full((G, wp_pad.shape[1])),
        out_shape=jax.ShapeDtypeStruct((G, wp_pad.shape[1]), f32),
    )

    h = h0
    vn = vn0
    for l in range(NL):
        w1, c1, w2, c2, bond = layer_w[l]
        hl = hl_call(h, bcol, vn)
        t_tab = tt_call(hl, bond.reshape(BV, 1, DP)).reshape(TROWS, DP)
        agg = edge_pass(t_tab, gidx_b, dl_b, cnt_b, zeros_hbm)
        h, ss_next = post_call(l == NL - 1)(hl, agg, brow, w1, c1, w2, c2)
        if l < NL - 1:
            vw1, vc1, vw2, vc2 = vn_w[l]
            vn = vn_call(ss_h, cnt_g, vn, vw1, vc1, vw2, vc2)
        ss_h = ss_next

    return head_call(ss_h, cnt_g, wp_pad, bp)


# speedup vs baseline: 7.4064x; 1.1894x over previous
import functools

import jax
import jax.numpy as jnp
from jax import lax
from jax.experimental import pallas as pl
from jax.experimental.pallas import tpu as pltpu
from jax.experimental.pallas import tpu_sc as plsc

N = 50000
E = 800000
D = 100
DP = 128
G = 512
NL = 5
AV = 119
BV = 5

BN = 1024
NB = 49
BNT = 7168
NBT = 7
N_PAD = NB * BN
NCHUNK = 4
CHUNK = N_PAD // NCHUNK
AGG_ROWS = 12672
TRASH0 = CHUNK
NW = 32
SHARE = 25088
E_PAD = NW * SHARE
CAP = SHARE
TROWS = BV * N_PAD
OB = 1024
SB = 64
EB = 4096
SSROWS = 640
SSB = 112
STG = 4352

_HI = lax.Precision.HIGHEST


def _bucketize_body(src_ref, dst_ref, attr_ref, gidx_ref, dl_ref, cnt_ref,
                    sbuf, dbuf, abuf,
                    stg0, std0, stg1, std1, stg2, std2, stg3, std3, cbuf):
    cidx = lax.axis_index("c")
    sidx = lax.axis_index("s")
    wid = sidx * 2 + cidx
    base = wid * SHARE
    lanes = lax.broadcasted_iota(jnp.int32, (16,), 0)
    stgs = ((stg0, std0), (stg1, std1), (stg2, std2), (stg3, std3))
    offs = [jnp.int32(0)] * NCHUNK
    fills = [jnp.int32(0)] * NCHUNK

    block_sizes = [EB] * (SHARE // EB)
    if SHARE % EB:
        block_sizes.append(SHARE % EB)
    boff = 0
    for bs in block_sizes:
        pltpu.sync_copy(src_ref.at[pl.ds(pl.multiple_of(base + boff, 8), bs)], sbuf.at[pl.ds(0, bs)])
        pltpu.sync_copy(dst_ref.at[pl.ds(pl.multiple_of(base + boff, 8), bs)], dbuf.at[pl.ds(0, bs)])
        pltpu.sync_copy(attr_ref.at[pl.ds(pl.multiple_of(base + boff, 8), bs)], abuf.at[pl.ds(0, bs)])

        def vbody(v, carry, boff=boff):
            sl = pl.ds(v * 16, 16)
            s = sbuf[sl]
            d = dbuf[sl]
            a = abuf[sl]
            pos = base + boff + v * 16 + lanes
            valid = pos < E
            c = ((d >= CHUNK).astype(jnp.int32)
                 + (d >= 2 * CHUNK).astype(jnp.int32)
                 + (d >= 3 * CHUNK).astype(jnp.int32))
            dl = d - c * CHUNK
            gx = a * N_PAD + s
            new = []
            for b in range(NCHUNK):
                ob = carry[b]
                m = jnp.logical_and(valid, c == b)
                pos = ob + plsc.cumsum(m.astype(jnp.int32)) - 1
                plsc.store_scatter(stgs[b][0], [pos], gx, mask=m)
                plsc.store_scatter(stgs[b][1], [pos], dl, mask=m)
                cnt = jnp.max(plsc.all_reduce_population_count(m))
                new.append(ob + cnt)
            return tuple(new)

        offs = list(lax.fori_loop(0, bs // 16, vbody, tuple(offs)))
        boff += bs

        for b in range(NCHUNK):
            sg, sd = stgs[b]
            rbase = (b * NW + wid) * CAP
            fill0 = fills[b]
            nfl = offs[b] >> 10

            def fl(j, _, sg=sg, sd=sd, rbase=rbase, fill0=fill0):
                pltpu.sync_copy(sg.at[pl.ds(j * 1024, 1024)],
                                gidx_ref.at[pl.ds(pl.multiple_of(rbase + fill0 + j * 1024, 8), 1024)])
                pltpu.sync_copy(sd.at[pl.ds(j * 1024, 1024)],
                                dl_ref.at[pl.ds(pl.multiple_of(rbase + fill0 + j * 1024, 8), 1024)])
                return 0

            lax.fori_loop(0, nfl, fl, 0)
            flushed = nfl << 10
            newoff = offs[b] - flushed
            nsh = (newoff + 15) >> 4

            def sh(j, _, sg=sg, sd=sd, flushed=flushed):
                sg[pl.ds(pl.multiple_of(j * 16, 8), 16)] = sg[pl.ds(pl.multiple_of(flushed + j * 16, 8), 16)]
                sd[pl.ds(pl.multiple_of(j * 16, 8), 16)] = sd[pl.ds(pl.multiple_of(flushed + j * 16, 8), 16)]
                return 0

            lax.fori_loop(0, nsh, sh, 0)
            fills[b] = fill0 + flushed
            offs[b] = newoff

    counts_vec = jnp.zeros((16,), jnp.int32)
    for b in range(NCHUNK):
        sg, sd = stgs[b]
        rbase = (b * NW + wid) * CAP
        fill0 = fills[b]
        total = fill0 + offs[b]
        nfl = (offs[b] + 15) >> 4

        def tf(j, _, sg=sg, sd=sd, rbase=rbase, fill0=fill0):
            pltpu.sync_copy(sg.at[pl.ds(pl.multiple_of(j * 16, 8), 16)],
                            gidx_ref.at[pl.ds(pl.multiple_of(rbase + fill0 + j * 16, 8), 16)])
            pltpu.sync_copy(sd.at[pl.ds(pl.multiple_of(j * 16, 8), 16)],
                            dl_ref.at[pl.ds(pl.multiple_of(rbase + fill0 + j * 16, 8), 16)])
            return 0

        lax.fori_loop(0, nfl, tf, 0)
        counts_vec = counts_vec + jnp.where(lanes == b, total, 0)
    cbuf[...] = counts_vec
    pltpu.sync_copy(cbuf, cnt_ref.at[pl.ds(pl.multiple_of(wid * 16, 8), 16)])


def _edge_body(t_ref, gidx_ref, dl_ref, cnt_ref, zeros_ref, agg_ref,
               gixb, dlb, dli2, rows0, rows1, cnts, aggS, sem, sems):
    cidx = lax.axis_index("c")
    sidx = lax.axis_index("s")
    lanes = lax.broadcasted_iota(jnp.int32, (16,), 0)
    rowbufs = (rows0, rows1)
    pltpu.sync_copy(cnt_ref, cnts)
    for p in range(2):
        c = cidx * 2 + p
        pltpu.sync_copy(zeros_ref.at[pl.ds(pl.multiple_of(sidx * (AGG_ROWS // 16), 8), AGG_ROWS // 16)],
                        aggS.at[pl.ds(pl.multiple_of(sidx * (AGG_ROWS // 16), 8), AGG_ROWS // 16)])
        plsc.subcore_barrier()
        for wsub in range(2):
            w = sidx * 2 + wsub
            cvec = cnts[pl.ds(pl.multiple_of(w * 16, 8), 16)]
            n = jnp.max(jnp.where(lanes == c, cvec, 0))
            rbase = (c * NW + w) * CAP
            nob = n >> 10

            def obody(j, _, rbase=rbase, n=n):
                obase = j * OB
                pltpu.sync_copy(gidx_ref.at[pl.ds(pl.multiple_of(rbase + obase, 8), OB)], gixb)
                pltpu.sync_copy(dl_ref.at[pl.ds(pl.multiple_of(rbase + obase, 8), OB)], dlb)
                for v in range(OB // 16):
                    sl = pl.ds(v * 16, 16)
                    posv = obase + v * 16 + lanes
                    mv = posv < n
                    gixb[sl] = jnp.where(mv, gixb[sl], posv & 2047)
                    dli2[v // (SB // 16), pl.ds((v % (SB // 16)) * 16, 16)] = jnp.where(
                        mv, dlb[sl], TRASH0 + (posv & 127))
                ns = OB // SB
                hg = [None] * ns
                hs = [None] * ns
                for sb in range(ns):
                    if sb >= 2:
                        hs[sb - 2].wait()
                    hg[sb] = pltpu.async_copy(
                        t_ref.at[gixb.at[pl.ds(pl.multiple_of(sb * SB, 8), SB)]],
                        rowbufs[sb & 1], sem)
                    if sb >= 1:
                        hg[sb - 1].wait()
                        hs[sb - 1] = pltpu.async_copy(
                            rowbufs[(sb - 1) & 1], aggS.at[dli2.at[sb - 1]], sems,
                            add=True)
                hg[ns - 1].wait()
                hs[ns - 1] = pltpu.async_copy(
                    rowbufs[(ns - 1) & 1], aggS.at[dli2.at[ns - 1]], sems, add=True)
                hs[ns - 2].wait()
                hs[ns - 1].wait()
                return 0

            lax.fori_loop(0, nob, obody, 0)
            ntail = (n - (nob << 10) + SB - 1) >> 6

            def tbody(t, _, rbase=rbase, n=n, nob=nob):
                boff = (nob << 10) + t * SB
                pltpu.sync_copy(gidx_ref.at[pl.ds(pl.multiple_of(rbase + boff, 8), SB)],
                                gixb.at[pl.ds(0, SB)])
                pltpu.sync_copy(dl_ref.at[pl.ds(pl.multiple_of(rbase + boff, 8), SB)],
                                dlb.at[pl.ds(0, SB)])
                for v in range(SB // 16):
                    sl = pl.ds(v * 16, 16)
                    posv = boff + v * 16 + lanes
                    mv = posv < n
                    gixb[sl] = jnp.where(mv, gixb[sl], posv & 2047)
                    dli2[0, pl.ds(v * 16, 16)] = jnp.where(
                        mv, dlb[sl], TRASH0 + (posv & 127))
                pltpu.async_copy(t_ref.at[gixb.at[pl.ds(0, SB)]], rows0, sem).wait()
                pltpu.sync_copy(rows0, aggS.at[dli2.at[0]], add=True)
                return 0

            lax.fori_loop(0, ntail, tbody, 0)
        plsc.subcore_barrier()
        pltpu.sync_copy(aggS.at[pl.ds(pl.multiple_of(sidx * (CHUNK // 16), 8), CHUNK // 16)],
                        agg_ref.at[pl.ds(pl.multiple_of(c * CHUNK + sidx * (CHUNK // 16), 8), CHUNK // 16)])
        plsc.subcore_barrier()


def _segsum_body(h_ref, bi_ref, zeros_ref, out_ref, idxb, rowsb, aggS, sem):
    cidx = lax.axis_index("c")
    sidx = lax.axis_index("s")
    base = cidx * (N_PAD // 2) + sidx * (N_PAD // NW)
    pltpu.sync_copy(zeros_ref.at[pl.ds(pl.multiple_of(sidx * (SSROWS // 16), 8), SSROWS // 16)],
                    aggS.at[pl.ds(pl.multiple_of(sidx * (SSROWS // 16), 8), SSROWS // 16)])
    plsc.subcore_barrier()
    for j in range(N_PAD // NW // SSB):
        pltpu.sync_copy(bi_ref.at[pl.ds(pl.multiple_of(base + j * SSB, 8), SSB)], idxb)
        pltpu.sync_copy(h_ref.at[pl.ds(pl.multiple_of(base + j * SSB, 8), SSB)], rowsb)
        pltpu.sync_copy(rowsb, aggS.at[idxb], add=True)
    plsc.subcore_barrier()
    pltpu.sync_copy(aggS.at[pl.ds(pl.multiple_of(sidx * 32, 8), 32)],
                    out_ref.at[pl.ds(pl.multiple_of(cidx * G + sidx * 32, 8), 32)])


def _embed_body(xcol_ref, brow_ref, emb_ref, h_ref, ss_ref, cnt_ref):
    ohx = (xcol_ref[...] == lax.broadcasted_iota(jnp.int32, (BN, DP), 1).astype(jnp.float32))
    ohx = ohx.astype(jnp.float32)
    h = jnp.dot(ohx, emb_ref[...], preferred_element_type=jnp.float32,
                precision=_HI)
    h_ref[...] = h
    ohbT = (brow_ref[...] == lax.broadcasted_iota(jnp.int32, (G, BN), 0).astype(jnp.float32))
    ohbT = ohbT.astype(jnp.float32)
    ss = jnp.dot(ohbT, h, preferred_element_type=jnp.float32, precision=_HI)
    cnt = jnp.sum(ohbT, axis=1, keepdims=True)

    @pl.when(pl.program_id(0) == 0)
    def _():
        ss_ref[...] = ss
        cnt_ref[...] = cnt

    @pl.when(pl.program_id(0) != 0)
    def _():
        ss_ref[...] += ss
        cnt_ref[...] += cnt


def _hl_body(h_ref, bcol_ref, vn_ref, hl_ref):
    ohb = (bcol_ref[...] == lax.broadcasted_iota(jnp.int32, (BN, G), 1).astype(jnp.float32))
    ohb = ohb.astype(jnp.float32)
    hl_ref[...] = h_ref[...] + jnp.dot(ohb, vn_ref[...],
                                       preferred_element_type=jnp.float32,
                                       precision=_HI)


def _tt_body(hl_ref, bond_ref, t_ref):
    t_ref[0] = jnp.maximum(hl_ref[...] + bond_ref[0], 0.0)


def _post_body(hl_ref, agg_ref, w1_ref, c1_ref, w2_ref, c2_ref, h_ref, *, last):
    z = hl_ref[...] + agg_ref[...]
    z1 = jnp.maximum(jnp.dot(z, w1_ref[...],
                             preferred_element_type=jnp.float32,
                             precision=_HI) + c1_ref[...], 0.0)
    h2 = jnp.dot(z1, w2_ref[...], preferred_element_type=jnp.float32,
                 precision=_HI) + c2_ref[...]
    if not last:
        h2 = jnp.maximum(h2, 0.0)
    h_ref[...] = h2


def _vn_body(ss_ref, cnt_ref, vn_ref, w1_ref, c1_ref, w2_ref, c2_ref, out_ref):
    vt = ss_ref[0:G] + ss_ref[G:2 * G] + (cnt_ref[...] + 1.0) * vn_ref[...]
    t = jnp.maximum(jnp.dot(vt, w1_ref[...],
                            preferred_element_type=jnp.float32,
                            precision=_HI) + c1_ref[...], 0.0)
    out_ref[...] = jnp.maximum(jnp.dot(t, w2_ref[...],
                                       preferred_element_type=jnp.float32,
                                       precision=_HI) + c2_ref[...], 0.0)


def _head_body(ss_ref, cnt_ref, wp_ref, bp_ref, out_ref):
    hg = (ss_ref[0:G] + ss_ref[G:2 * G]) / jnp.maximum(cnt_ref[...], 1.0)
    out_ref[...] = jnp.dot(hg, wp_ref[...], preferred_element_type=jnp.float32,
                           precision=_HI) + bp_ref[...]


def _pad2(w, rows, cols):
    out = jnp.zeros((rows, cols), jnp.float32)
    return out.at[:w.shape[0], :w.shape[1]].set(w)


def _pad1(v, n):
    out = jnp.zeros((n,), jnp.float32)
    return out.at[:v.shape[0]].set(v)


def kernel(x, edge_index, edge_attr, batch, params):
    f32 = jnp.float32
    i32 = jnp.int32

    src = edge_index[0].astype(i32)
    dst = edge_index[1].astype(i32)
    attr = edge_attr.astype(i32)
    pad_e = E_PAD - E
    src_p = jnp.pad(src, (0, pad_e))
    dst_p = jnp.pad(dst, (0, pad_e))
    attr_p = jnp.pad(attr, (0, pad_e))

    pad_n = N_PAD - N
    xcol = jnp.pad(x.astype(f32), (0, pad_n),
                   constant_values=99999.0).reshape(N_PAD, 1)
    bflat = jnp.pad(batch.astype(f32), (0, pad_n), constant_values=99999.0)
    batchi = jnp.concatenate([batch.astype(jnp.int32),
                              G + (jnp.arange(pad_n, dtype=jnp.int32) & 127)])
    bcol = bflat.reshape(N_PAD, 1)
    brow = bflat.reshape(1, N_PAD)

    atom_pad = _pad2(params['atom_emb'], DP, DP)
    vn0 = jnp.tile(_pad1(params['vn_emb'], DP)[None, :], (G, 1))
    wp_pad = _pad2(params['wp'], DP, params['wp'].shape[1])
    bp = params['bp'][None, :]

    layer_w = []
    for lp in params['layers']:
        w1 = _pad2(lp['w1'] * lp['g1'][None, :], DP, 2 * D)
        c1 = (lp['b1'] * lp['g1'] + lp['be1'])[None, :]
        w2 = _pad2(lp['w2'] * lp['gn'][None, :], 2 * D, DP)
        c2 = _pad1(lp['b2'] * lp['gn'] + lp['bn'], DP)[None, :]
        bond = _pad2(lp['bond_emb'], BV, DP)
        layer_w.append((w1, c1, w2, c2, bond))
    vn_w = []
    for vp in params['vn_mlps']:
        w1 = _pad2(vp['w1'] * vp['g1'][None, :], DP, 2 * D)
        c1 = (vp['b1'] * vp['g1'] + vp['be1'])[None, :]
        w2 = _pad2(vp['w2'] * vp['g2'][None, :], 2 * D, DP)
        c2 = _pad1(vp['b2'] * vp['g2'] + vp['be2'], DP)[None, :]
        vn_w.append((w1, c1, w2, c2))

    zeros_hbm = jnp.zeros((AGG_ROWS, DP), f32)

    _sc_mesh = plsc.VectorSubcoreMesh(core_axis_name="c", subcore_axis_name="s")
    sc_params = pltpu.CompilerParams(needs_layout_passes=False)
    bucketize = functools.partial(
        pl.kernel,
        mesh=_sc_mesh,
        compiler_params=sc_params,
        out_type=[
            jax.ShapeDtypeStruct((NCHUNK * NW * CAP,), i32),
            jax.ShapeDtypeStruct((NCHUNK * NW * CAP,), i32),
            jax.ShapeDtypeStruct((NW * 16,), i32),
        ],
        scratch_types=(
            [pltpu.VMEM((EB,), i32) for _ in range(3)]
            + [pltpu.VMEM((STG,), i32) for _ in range(8)]
            + [pltpu.VMEM((16,), i32)]
        ),
    )(_bucketize_body)
    gidx_b, dl_b, cnt_b = bucketize(src_p, dst_p, attr_p)

    edge_pass = functools.partial(
        pl.kernel,
        mesh=_sc_mesh,
        compiler_params=sc_params,
        out_type=jax.ShapeDtypeStruct((N_PAD, DP), f32),
        scratch_types=[
            pltpu.VMEM((OB,), i32),
            pltpu.VMEM((OB,), i32),
            pltpu.VMEM((OB // SB, SB), i32),
            pltpu.VMEM((SB, DP), f32),
            pltpu.VMEM((SB, DP), f32),
            pltpu.VMEM((NW * 16,), i32),
            pltpu.VMEM_SHARED((AGG_ROWS, DP), f32),
            pltpu.SemaphoreType.DMA,
            pltpu.SemaphoreType.DMA,
        ],
    )(_edge_body)

    segsum_call = functools.partial(
        pl.kernel,
        mesh=_sc_mesh,
        compiler_params=sc_params,
        out_type=jax.ShapeDtypeStruct((2 * G, DP), f32),
        scratch_types=[
            pltpu.VMEM((SSB,), i32),
            pltpu.VMEM((SSB, DP), f32),
            pltpu.VMEM_SHARED((SSROWS, DP), f32),
            pltpu.SemaphoreType.DMA,
        ],
    )(_segsum_body)

    vspec = pl.BlockSpec((BN, DP), lambda i: (i, 0))
    cspec = pl.BlockSpec((BN, 1), lambda i: (i, 0))
    rspec = pl.BlockSpec((1, BN), lambda i: (0, i))
    gspec = pl.BlockSpec((G, DP), lambda i: (0, 0))
    g1spec = pl.BlockSpec((G, 1), lambda i: (0, 0))

    def full(shape):
        return pl.BlockSpec(shape, lambda *a: tuple(0 for _ in shape))

    h0, ss_h, cnt_g = pl.pallas_call(
        _embed_body,
        grid=(NB,),
        in_specs=[cspec, rspec, full((DP, DP))],
        out_specs=[vspec, gspec, g1spec],
        out_shape=[
            jax.ShapeDtypeStruct((N_PAD, DP), f32),
            jax.ShapeDtypeStruct((G, DP), f32),
            jax.ShapeDtypeStruct((G, 1), f32),
        ],
    )(xcol, brow, atom_pad)

    hl_call = pl.pallas_call(
        _hl_body,
        grid=(NB,),
        in_specs=[vspec, cspec, gspec],
        out_specs=vspec,
        out_shape=jax.ShapeDtypeStruct((N_PAD, DP), f32),
    )

    tt_call = pl.pallas_call(
        _tt_body,
        grid=(BV, NBT),
        in_specs=[pl.BlockSpec((BNT, DP), lambda a, i: (i, 0)),
                  pl.BlockSpec((1, 1, DP), lambda a, i: (a, 0, 0))],
        out_specs=pl.BlockSpec((1, BNT, DP), lambda a, i: (a, i, 0)),
        out_shape=jax.ShapeDtypeStruct((BV, N_PAD, DP), f32),
    )

    def post_call(last):
        return pl.pallas_call(
            functools.partial(_post_body, last=last),
            grid=(NB,),
            in_specs=[vspec, vspec, full((DP, 2 * D)), full((1, 2 * D)),
                      full((2 * D, DP)), full((1, DP))],
            out_specs=vspec,
            out_shape=jax.ShapeDtypeStruct((N_PAD, DP), f32),
        )

    vn_call = pl.pallas_call(
        _vn_body,
        in_specs=[full((2 * G, DP)), full((G, 1)), full((G, DP)),
                  full((DP, 2 * D)), full((1, 2 * D)),
                  full((2 * D, DP)), full((1, DP))],
        out_specs=full((G, DP)),
        out_shape=jax.ShapeDtypeStruct((G, DP), f32),
    )

    head_call = pl.pallas_call(
        _head_body,
        in_specs=[full((2 * G, DP)), full((G, 1)),
                  full((DP, wp_pad.shape[1])), full((1, wp_pad.shape[1]))],
        out_specs=full((G, wp_pad.shape[1])),
        out_shape=jax.ShapeDtypeStruct((G, wp_pad.shape[1]), f32),
    )

    h = h0
    vn = vn0
    ss_h2 = jnp.concatenate([ss_h, jnp.zeros_like(ss_h)], axis=0)
    for l in range(NL):
        w1, c1, w2, c2, bond = layer_w[l]
        hl = hl_call(h, bcol, vn)
        t_tab = tt_call(hl, bond.reshape(BV, 1, DP)).reshape(TROWS, DP)
        agg = edge_pass(t_tab, gidx_b, dl_b, cnt_b, zeros_hbm)
        h = post_call(l == NL - 1)(hl, agg, w1, c1, w2, c2)
        ss_next = segsum_call(h, batchi, zeros_hbm)
        if l < NL - 1:
            vw1, vc1, vw2, vc2 = vn_w[l]
            vn = vn_call(ss_h2, cnt_g, vn, vw1, vc1, vw2, vc2)
        ss_h2 = ss_next

    return head_call(ss_h2, cnt_g, wp_pad, bp)
